# baseline (speedup 1.0000x reference)
import functools

import jax
import jax.numpy as jnp
from jax import lax
from jax.experimental import pallas as pl
from jax.experimental.pallas import tpu as pltpu
from jax.experimental.pallas import tpu_sc as plsc

B, T, D = 16, 2048, 256
VOCAB = 100000

_NC = 2
_NS = 16
_LANES = 16
_NW = _NC * _NS
_N_IDX = B * T
_CHUNK = _N_IDX // _NW


def _sc_gather(weights, idx_flat):
    mesh = plsc.VectorSubcoreMesh(core_axis_name="c", subcore_axis_name="s")

    @functools.partial(
        pl.kernel,
        mesh=mesh,
        out_type=jax.ShapeDtypeStruct((_N_IDX,), jnp.float32),
        scratch_types=[
            pltpu.VMEM((VOCAB,), jnp.float32),
            pltpu.VMEM((_CHUNK,), jnp.int32),
            pltpu.VMEM((_CHUNK,), jnp.float32),
        ],
        compiler_params=pltpu.CompilerParams(needs_layout_passes=False),
    )
    def gather_kernel(w_hbm, idx_hbm, out_hbm, wtab_v, idx_v, out_v):
        wid = lax.axis_index("s") * _NC + lax.axis_index("c")
        base = wid * _CHUNK
        pltpu.sync_copy(w_hbm, wtab_v)
        pltpu.sync_copy(idx_hbm.at[pl.ds(base, _CHUNK)], idx_v)

        def body(i, carry):
            off = i * _LANES
            idx16 = idx_v[pl.ds(off, _LANES)]
            out_v[pl.ds(off, _LANES)] = plsc.load_gather(wtab_v, [idx16])
            return carry

        lax.fori_loop(0, _CHUNK // _LANES, body, 0, unroll=4)
        pltpu.sync_copy(out_v, out_hbm.at[pl.ds(base, _CHUNK)])

    return gather_kernel(weights, idx_flat)


def _tc_body(len_ref, vs_ref, w_ref, y_ref, yh_ref):
    b = pl.program_id(0)
    length = len_ref[b]
    pos = lax.broadcasted_iota(jnp.int32, (1, T), 1)
    maskf = (pos < length).astype(jnp.float32)
    w_row = w_ref[0, :, :] * maskf
    a = jnp.concatenate([maskf, w_row], axis=0)
    vs = vs_ref[0, :, :]
    acc = jnp.dot(a, vs, preferred_element_type=jnp.float32)
    s = acc[0:1, :]
    denom = jnp.sqrt(jnp.sum(jnp.abs(s)))
    y_ref[0, :, :] = s / denom
    yh_ref[0, :, :] = acc[1:2, :]


def kernel(vector_sequence, sentence_length, word_sequence, weights):
    idx_flat = word_sequence.reshape(-1).astype(jnp.int32)
    w_tok = _sc_gather(weights, idx_flat)
    w3 = w_tok.reshape(B, 1, T)
    lens = sentence_length.astype(jnp.int32)

    y3, yh3 = pl.pallas_call(
        _tc_body,
        grid=(B,),
        in_specs=[
            pl.BlockSpec(memory_space=pltpu.SMEM),
            pl.BlockSpec((1, T, D), lambda b: (b, 0, 0)),
            pl.BlockSpec((1, 1, T), lambda b: (b, 0, 0)),
        ],
        out_specs=[
            pl.BlockSpec((1, 1, D), lambda b: (b, 0, 0)),
            pl.BlockSpec((1, 1, D), lambda b: (b, 0, 0)),
        ],
        out_shape=[
            jax.ShapeDtypeStruct((B, 1, D), jnp.float32),
            jax.ShapeDtypeStruct((B, 1, D), jnp.float32),
        ],
    )(lens, vector_sequence, w3)
    return y3.reshape(B, D), yh3.reshape(B, D)
